# Initial kernel scaffold; baseline (speedup 1.0000x reference)
import jax
import jax.numpy as jnp
from jax.experimental import pallas as pl


def kernel(x, edge_index, proj_W, proj_b, ln_g, ln_b, W0, b0, g0, be0, W1, b1, g1, be1, W2, b2, g2, be2):
    raise NotImplementedError("write your pallas kernel here")


# speedup vs baseline: 5.2689x; 5.2689x over previous
import functools

import jax
import jax.numpy as jnp
from jax import lax
from jax.experimental import pallas as pl
from jax.experimental.pallas import tpu as pltpu
from jax.experimental.pallas import tpu_sc as plsc

N = 10000
E = 160000
DIN = 256
D = 512
QW = 128
NQ = D // QW
NC, NS = 2, 16
EB = 128
NROW = E // EB
NROWP = 1280
RPT_SC = NROWP // NS
RPT_DEG = NROWP // (NC * NS)
NPAD = 16
OWN = 624
NEXTRA = N - OWN * NS
ZR = 104

_mesh = plsc.VectorSubcoreMesh(
    core_axis_name="c", subcore_axis_name="s", num_cores=NC, num_subcores=NS)


def _fill_rows(ref, nrows, width, value):
  def body(i, _):
    for k in range(width // 16):
      ref[i, pl.ds(k * 16, 16)] = jnp.full((16,), value, ref.dtype)
    return 0
  lax.fori_loop(0, nrows, body, 0)


def _zero_own_rows(zb_v, acc_sh, s):
  for k in range(OWN // ZR):
    pltpu.sync_copy(zb_v, acc_sh.at[pl.ds(s * OWN + k * ZR, ZR)])

  @pl.when(s < NEXTRA // 8)
  def _():
    pltpu.sync_copy(zb_v.at[pl.ds(0, 8)],
                    acc_sh.at[pl.ds(OWN * NS + s * 8, 8)])


def _write_own_rows(acc_sh, out_view, s):
  pltpu.sync_copy(acc_sh.at[pl.ds(s * OWN, OWN)],
                  out_view.at[pl.ds(s * OWN, OWN)])

  @pl.when(s < NEXTRA // 8)
  def _():
    pltpu.sync_copy(acc_sh.at[pl.ds(OWN * NS + s * 8, 8)],
                    out_view.at[pl.ds(OWN * NS + s * 8, 8)])


def _sc_deg_body(dstb_hbm, out_hbm, dst_v, ones_v, zb_v, acc_sh):
  c = lax.axis_index("c")
  s = lax.axis_index("s")
  wid = c * NS + s
  base = wid * RPT_DEG
  pltpu.sync_copy(dstb_hbm.at[pl.ds(base, RPT_DEG)], dst_v)
  _fill_rows(ones_v, EB, 16, 1.0)
  _fill_rows(zb_v, ZR, 16, 0.0)
  _zero_own_rows(zb_v, acc_sh, s)
  plsc.subcore_barrier()

  def body(j, _):
    pltpu.sync_copy(ones_v, acc_sh.at[dst_v.at[j]], add=True)
    return 0
  lax.fori_loop(0, RPT_DEG, body, 0)
  plsc.subcore_barrier()
  _write_own_rows(acc_sh, out_hbm.at[c], s)


_sc_deg = pl.kernel(
    _sc_deg_body,
    out_type=jax.ShapeDtypeStruct((NC, N, 16), jnp.float32),
    mesh=_mesh,
    scratch_types=[
        pltpu.VMEM((RPT_DEG, EB), jnp.int32),
        pltpu.VMEM((EB, 16), jnp.float32),
        pltpu.VMEM((ZR, 16), jnp.float32),
        pltpu.VMEM_SHARED((N + NPAD, 16), jnp.float32),
    ],
)


def _sc_scatter_body(hs_hbm, srcb_hbm, dstb_hbm, out_hbm,
                     src_v, dst_v, rows_v, zb_v, acc_sh, sem):
  c = lax.axis_index("c")
  s = lax.axis_index("s")
  base = s * RPT_SC
  pltpu.sync_copy(srcb_hbm.at[pl.ds(base, RPT_SC)], src_v)
  pltpu.sync_copy(dstb_hbm.at[pl.ds(base, RPT_SC)], dst_v)
  _fill_rows(zb_v, ZR, QW, 0.0)

  for ql in range(NQ // NC):
    q = c * (NQ // NC) + ql
    _zero_own_rows(zb_v, acc_sh, s)
    plsc.subcore_barrier()
    hsv = hs_hbm.at[q]

    def body(j, _):
      cp = pltpu.make_async_copy(hsv.at[src_v.at[j]], rows_v, sem)
      cp.start()
      cp.wait()
      pltpu.sync_copy(rows_v, acc_sh.at[dst_v.at[j]], add=True)
      return 0
    lax.fori_loop(0, RPT_SC, body, 0)
    plsc.subcore_barrier()
    _write_own_rows(acc_sh, out_hbm.at[q], s)


_sc_scatter = pl.kernel(
    _sc_scatter_body,
    out_type=jax.ShapeDtypeStruct((NQ, N, QW), jnp.float32),
    mesh=_mesh,
    scratch_types=[
        pltpu.VMEM((RPT_SC, EB), jnp.int32),
        pltpu.VMEM((RPT_SC, EB), jnp.int32),
        pltpu.VMEM((EB, QW), jnp.float32),
        pltpu.VMEM((ZR, QW), jnp.float32),
        pltpu.VMEM_SHARED((N + NPAD, QW), jnp.float32),
        pltpu.SemaphoreType.DMA,
    ],
)

R = 1000
GRID = N // R


def _gelu(x):
  return x * 0.5 * (1.0 + lax.erf(x * 0.7071067811865475))


def _tc_a_body(x_ref, pw_ref, pb_ref, lg_ref, lb_ref, w0_ref, deg_ref,
               h0_ref, hs0_ref, dis_ref):
  xb = x_ref[...]
  h = jnp.dot(xb, pw_ref[...], preferred_element_type=jnp.float32) + pb_ref[...]
  mu = jnp.mean(h, axis=1, keepdims=True)
  var = jnp.mean((h - mu) ** 2, axis=1, keepdims=True)
  hn = (h - mu) * lax.rsqrt(var + 1e-5) * lg_ref[...] + lb_ref[...]
  g = _gelu(hn)
  d = deg_ref[...]
  deg = d[0, :, 0:1] + d[1, :, 0:1] + 1.0
  dis = lax.rsqrt(deg)
  dis_ref[...] = jnp.broadcast_to(dis, (R, QW))
  z = jnp.dot(g, w0_ref[...], preferred_element_type=jnp.float32)
  for q in range(NQ):
    h0_ref[q, :, :] = g[:, q * QW:(q + 1) * QW]
    hs0_ref[q, :, :] = dis * z[:, q * QW:(q + 1) * QW]


_tc_a = pl.pallas_call(
    _tc_a_body,
    grid=(GRID,),
    in_specs=[
        pl.BlockSpec((R, DIN), lambda i: (i, 0)),
        pl.BlockSpec((DIN, D), lambda i: (0, 0)),
        pl.BlockSpec((1, D), lambda i: (0, 0)),
        pl.BlockSpec((1, D), lambda i: (0, 0)),
        pl.BlockSpec((1, D), lambda i: (0, 0)),
        pl.BlockSpec((D, D), lambda i: (0, 0)),
        pl.BlockSpec((NC, R, 16), lambda i: (0, i, 0)),
    ],
    out_specs=[
        pl.BlockSpec((NQ, R, QW), lambda i: (0, i, 0)),
        pl.BlockSpec((NQ, R, QW), lambda i: (0, i, 0)),
        pl.BlockSpec((R, QW), lambda i: (i, 0)),
    ],
    out_shape=[
        jax.ShapeDtypeStruct((NQ, N, QW), jnp.float32),
        jax.ShapeDtypeStruct((NQ, N, QW), jnp.float32),
        jax.ShapeDtypeStruct((N, QW), jnp.float32),
    ],
)


def _layer_core(acc_ref, hs_ref, res_ref, dis_ref, b_ref, g_ref, be_ref):
  dis = dis_ref[...]
  acc = acc_ref[...]
  hs = hs_ref[...]
  res = res_ref[...]
  rs = []
  for q in range(NQ):
    cols = pl.ds(q * QW, QW)
    u = dis * (acc[q] + hs[q]) + b_ref[0, cols]
    rs.append(_gelu(u) + res[q])
  mu = sum(jnp.sum(r, axis=1, keepdims=True) for r in rs) / D
  var = sum(jnp.sum((r - mu) ** 2, axis=1, keepdims=True) for r in rs) / D
  inv = lax.rsqrt(var + 1e-5)
  ys = []
  for q in range(NQ):
    cols = pl.ds(q * QW, QW)
    ys.append((rs[q] - mu) * inv * g_ref[0, cols] + be_ref[0, cols])
  return ys, dis


def _tc_b_body(acc_ref, hs_ref, res_ref, dis_ref, b_ref, g_ref, be_ref, wn_ref,
               hq_ref, hsn_ref):
  ys, dis = _layer_core(acc_ref, hs_ref, res_ref, dis_ref, b_ref, g_ref, be_ref)
  z = None
  for q in range(NQ):
    hq_ref[q, :, :] = ys[q]
    part = jnp.dot(ys[q], wn_ref[pl.ds(q * QW, QW), :],
                   preferred_element_type=jnp.float32)
    z = part if z is None else z + part
  for q in range(NQ):
    hsn_ref[q, :, :] = dis * z[:, q * QW:(q + 1) * QW]


_tc_b = pl.pallas_call(
    _tc_b_body,
    grid=(GRID,),
    in_specs=[
        pl.BlockSpec((NQ, R, QW), lambda i: (0, i, 0)),
        pl.BlockSpec((NQ, R, QW), lambda i: (0, i, 0)),
        pl.BlockSpec((NQ, R, QW), lambda i: (0, i, 0)),
        pl.BlockSpec((R, QW), lambda i: (i, 0)),
        pl.BlockSpec((1, D), lambda i: (0, 0)),
        pl.BlockSpec((1, D), lambda i: (0, 0)),
        pl.BlockSpec((1, D), lambda i: (0, 0)),
        pl.BlockSpec((D, D), lambda i: (0, 0)),
    ],
    out_specs=[
        pl.BlockSpec((NQ, R, QW), lambda i: (0, i, 0)),
        pl.BlockSpec((NQ, R, QW), lambda i: (0, i, 0)),
    ],
    out_shape=[
        jax.ShapeDtypeStruct((NQ, N, QW), jnp.float32),
        jax.ShapeDtypeStruct((NQ, N, QW), jnp.float32),
    ],
)


def _tc_c_body(acc_ref, hs_ref, res_ref, dis_ref, b_ref, g_ref, be_ref,
               h_ref, gs_ref):
  ys, _ = _layer_core(acc_ref, hs_ref, res_ref, dis_ref, b_ref, g_ref, be_ref)
  for q in range(NQ):
    h_ref[:, pl.ds(q * QW, QW)] = ys[q]
  part = jnp.concatenate(
      [jnp.sum(ys[q], axis=0, keepdims=True) for q in range(NQ)], axis=1)

  @pl.when(pl.program_id(0) == 0)
  def _():
    gs_ref[...] = jnp.zeros((1, D), jnp.float32)

  gs_ref[...] += part * (1.0 / N)


_tc_c = pl.pallas_call(
    _tc_c_body,
    grid=(GRID,),
    in_specs=[
        pl.BlockSpec((NQ, R, QW), lambda i: (0, i, 0)),
        pl.BlockSpec((NQ, R, QW), lambda i: (0, i, 0)),
        pl.BlockSpec((NQ, R, QW), lambda i: (0, i, 0)),
        pl.BlockSpec((R, QW), lambda i: (i, 0)),
        pl.BlockSpec((1, D), lambda i: (0, 0)),
        pl.BlockSpec((1, D), lambda i: (0, 0)),
        pl.BlockSpec((1, D), lambda i: (0, 0)),
    ],
    out_specs=[
        pl.BlockSpec((R, D), lambda i: (i, 0)),
        pl.BlockSpec((1, D), lambda i: (0, 0)),
    ],
    out_shape=[
        jax.ShapeDtypeStruct((N, D), jnp.float32),
        jax.ShapeDtypeStruct((1, D), jnp.float32),
    ],
)


def kernel(x, edge_index, proj_W, proj_b, ln_g, ln_b,
           W0, b0, g0, be0, W1, b1, g1, be1, W2, b2, g2, be2):
  npad = NROWP * EB - E
  srcb = jnp.concatenate(
      [edge_index[0], jnp.zeros((npad,), jnp.int32)]).reshape(NROWP, EB)
  dstb = jnp.concatenate(
      [edge_index[1], jnp.full((npad,), N, jnp.int32)]).reshape(NROWP, EB)
  deg16 = _sc_deg(dstb)
  h0q, hs0q, dis2 = _tc_a(x, proj_W, proj_b.reshape(1, D), ln_g.reshape(1, D),
                          ln_b.reshape(1, D), W0, deg16)
  hq, hsq = h0q, hs0q
  layer_params = ((b0, g0, be0), (b1, g1, be1), (b2, g2, be2))
  next_w = (W1, W2)
  for i in range(2):
    accq = _sc_scatter(hsq, srcb, dstb)
    b, g, be = layer_params[i]
    hq, hsq = _tc_b(accq, hsq, hq, dis2, b.reshape(1, D), g.reshape(1, D),
                    be.reshape(1, D), next_w[i])
  accq = _sc_scatter(hsq, srcb, dstb)
  b, g, be = layer_params[2]
  h, gs = _tc_c(accq, hsq, hq, dis2, b.reshape(1, D), g.reshape(1, D),
                be.reshape(1, D))
  return h, gs.reshape(D)


# speedup vs baseline: 6.2851x; 1.1929x over previous
import functools

import jax
import jax.numpy as jnp
from jax import lax
from jax.experimental import pallas as pl
from jax.experimental.pallas import tpu as pltpu
from jax.experimental.pallas import tpu_sc as plsc

N = 10000
E = 160000
DIN = 256
D = 512
QW = 128
NQ = D // QW
NC, NS = 2, 16
EB = 128
NROW = E // EB
NROWP = 1280
RPT_SC = NROWP // NS
RPT_DEG = NROWP // (NC * NS)
NPAD = 16
OWN = 624
NEXTRA = N - OWN * NS
ZR = 104
ZB = 24
IDXC = 40

_mesh = plsc.VectorSubcoreMesh(
    core_axis_name="c", subcore_axis_name="s", num_cores=NC, num_subcores=NS)


def _fill_rows(ref, nrows, width, value):
  def body(i, _):
    for k in range(width // 16):
      ref[i, pl.ds(k * 16, 16)] = jnp.full((16,), value, ref.dtype)
    return 0
  lax.fori_loop(0, nrows, body, 0)


def _zero_own_rows(zb_v, acc_sh, s):
  chunk = zb_v.shape[0]
  for k in range(OWN // chunk):
    pltpu.sync_copy(zb_v, acc_sh.at[pl.ds(s * OWN + k * chunk, chunk)])

  @pl.when(s < NEXTRA // 8)
  def _():
    pltpu.sync_copy(zb_v.at[pl.ds(0, 8)],
                    acc_sh.at[pl.ds(OWN * NS + s * 8, 8)])


def _write_own_rows(acc_sh, out_view, s):
  pltpu.sync_copy(acc_sh.at[pl.ds(s * OWN, OWN)],
                  out_view.at[pl.ds(s * OWN, OWN)])

  @pl.when(s < NEXTRA // 8)
  def _():
    pltpu.sync_copy(acc_sh.at[pl.ds(OWN * NS + s * 8, 8)],
                    out_view.at[pl.ds(OWN * NS + s * 8, 8)])


def _sc_deg_body(dstb_hbm, out_hbm, dst_v, ones_v, zb_v, acc_sh):
  c = lax.axis_index("c")
  s = lax.axis_index("s")
  wid = c * NS + s
  base = wid * RPT_DEG
  pltpu.sync_copy(dstb_hbm.at[pl.ds(base, RPT_DEG)], dst_v)
  _fill_rows(ones_v, EB, 16, 1.0)
  _fill_rows(zb_v, ZR, 16, 0.0)
  _zero_own_rows(zb_v, acc_sh, s)
  plsc.subcore_barrier()

  def body(j, _):
    pltpu.sync_copy(ones_v, acc_sh.at[dst_v.at[j]], add=True)
    return 0
  lax.fori_loop(0, RPT_DEG, body, 0)
  plsc.subcore_barrier()
  _write_own_rows(acc_sh, out_hbm.at[c], s)


_sc_deg = pl.kernel(
    _sc_deg_body,
    out_type=jax.ShapeDtypeStruct((NC, N, 16), jnp.float32),
    mesh=_mesh,
    scratch_types=[
        pltpu.VMEM((RPT_DEG, EB), jnp.int32),
        pltpu.VMEM((EB, 16), jnp.float32),
        pltpu.VMEM((ZR, 16), jnp.float32),
        pltpu.VMEM_SHARED((N + NPAD, 16), jnp.float32),
    ],
)


def _sc_scatter_body(hs_hbm, srcb_hbm, dstb_hbm, out_hbm,
                     src_v, dst_v, rows_a, rows_b, zb_v, acc_sh, sem_a, sem_b):
  c = lax.axis_index("c")
  s = lax.axis_index("s")
  _fill_rows(zb_v, ZB, QW, 0.0)

  for ql in range(NQ // NC):
    q = c * (NQ // NC) + ql
    _zero_own_rows(zb_v, acc_sh, s)
    plsc.subcore_barrier()
    hsv = hs_hbm.at[q]

    for half in range(RPT_SC // IDXC):
      base = s * RPT_SC + half * IDXC
      pltpu.sync_copy(srcb_hbm.at[pl.ds(base, IDXC)], src_v)
      pltpu.sync_copy(dstb_hbm.at[pl.ds(base, IDXC)], dst_v)

      def _gather_a(j):
        return pltpu.make_async_copy(hsv.at[src_v.at[j]], rows_a, sem_a)

      def _gather_b(j):
        return pltpu.make_async_copy(hsv.at[src_v.at[j]], rows_b, sem_b)

      _gather_a(0).start()

      def body(i, _):
        j0 = 2 * i
        j1 = j0 + 1

        @pl.when(j1 < IDXC)
        def _():
          _gather_b(j1).start()
        _gather_a(j0).wait()
        pltpu.sync_copy(rows_a, acc_sh.at[dst_v.at[j0]], add=True)

        @pl.when(j0 + 2 < IDXC)
        def _():
          _gather_a(j0 + 2).start()
        _gather_b(j1).wait()
        pltpu.sync_copy(rows_b, acc_sh.at[dst_v.at[j1]], add=True)
        return 0
      lax.fori_loop(0, IDXC // 2, body, 0)

    plsc.subcore_barrier()
    _write_own_rows(acc_sh, out_hbm.at[q], s)


_sc_scatter = pl.kernel(
    _sc_scatter_body,
    out_type=jax.ShapeDtypeStruct((NQ, N, QW), jnp.float32),
    mesh=_mesh,
    scratch_types=[
        pltpu.VMEM((IDXC, EB), jnp.int32),
        pltpu.VMEM((IDXC, EB), jnp.int32),
        pltpu.VMEM((EB, QW), jnp.float32),
        pltpu.VMEM((EB, QW), jnp.float32),
        pltpu.VMEM((ZB, QW), jnp.float32),
        pltpu.VMEM_SHARED((N + NPAD, QW), jnp.float32),
        pltpu.SemaphoreType.DMA,
        pltpu.SemaphoreType.DMA,
    ],
)

R = 1000
GRID = N // R


def _gelu(x):
  return x * 0.5 * (1.0 + lax.erf(x * 0.7071067811865475))


def _tc_a_body(x_ref, pw_ref, pb_ref, lg_ref, lb_ref, w0_ref, deg_ref,
               h0_ref, hs0_ref, dis_ref):
  xb = x_ref[...]
  h = jnp.dot(xb, pw_ref[...], preferred_element_type=jnp.float32) + pb_ref[...]
  mu = jnp.mean(h, axis=1, keepdims=True)
  var = jnp.mean((h - mu) ** 2, axis=1, keepdims=True)
  hn = (h - mu) * lax.rsqrt(var + 1e-5) * lg_ref[...] + lb_ref[...]
  g = _gelu(hn)
  d = deg_ref[...]
  deg = d[0, :, 0:1] + d[1, :, 0:1] + 1.0
  dis = lax.rsqrt(deg)
  dis_ref[...] = jnp.broadcast_to(dis, (R, QW))
  z = jnp.dot(g, w0_ref[...], preferred_element_type=jnp.float32)
  for q in range(NQ):
    h0_ref[q, :, :] = g[:, q * QW:(q + 1) * QW]
    hs0_ref[q, :, :] = dis * z[:, q * QW:(q + 1) * QW]


_tc_a = pl.pallas_call(
    _tc_a_body,
    grid=(GRID,),
    in_specs=[
        pl.BlockSpec((R, DIN), lambda i: (i, 0)),
        pl.BlockSpec((DIN, D), lambda i: (0, 0)),
        pl.BlockSpec((1, D), lambda i: (0, 0)),
        pl.BlockSpec((1, D), lambda i: (0, 0)),
        pl.BlockSpec((1, D), lambda i: (0, 0)),
        pl.BlockSpec((D, D), lambda i: (0, 0)),
        pl.BlockSpec((NC, R, 16), lambda i: (0, i, 0)),
    ],
    out_specs=[
        pl.BlockSpec((NQ, R, QW), lambda i: (0, i, 0)),
        pl.BlockSpec((NQ, R, QW), lambda i: (0, i, 0)),
        pl.BlockSpec((R, QW), lambda i: (i, 0)),
    ],
    out_shape=[
        jax.ShapeDtypeStruct((NQ, N, QW), jnp.float32),
        jax.ShapeDtypeStruct((NQ, N, QW), jnp.float32),
        jax.ShapeDtypeStruct((N, QW), jnp.float32),
    ],
)


def _layer_core(acc_ref, hs_ref, res_ref, dis_ref, b_ref, g_ref, be_ref):
  dis = dis_ref[...]
  acc = acc_ref[...]
  hs = hs_ref[...]
  res = res_ref[...]
  rs = []
  for q in range(NQ):
    cols = pl.ds(q * QW, QW)
    u = dis * (acc[q] + hs[q]) + b_ref[0, cols]
    rs.append(_gelu(u) + res[q])
  mu = sum(jnp.sum(r, axis=1, keepdims=True) for r in rs) / D
  var = sum(jnp.sum((r - mu) ** 2, axis=1, keepdims=True) for r in rs) / D
  inv = lax.rsqrt(var + 1e-5)
  ys = []
  for q in range(NQ):
    cols = pl.ds(q * QW, QW)
    ys.append((rs[q] - mu) * inv * g_ref[0, cols] + be_ref[0, cols])
  return ys, dis


def _tc_b_body(acc_ref, hs_ref, res_ref, dis_ref, b_ref, g_ref, be_ref, wn_ref,
               hq_ref, hsn_ref):
  ys, dis = _layer_core(acc_ref, hs_ref, res_ref, dis_ref, b_ref, g_ref, be_ref)
  z = None
  for q in range(NQ):
    hq_ref[q, :, :] = ys[q]
    part = jnp.dot(ys[q], wn_ref[pl.ds(q * QW, QW), :],
                   preferred_element_type=jnp.float32)
    z = part if z is None else z + part
  for q in range(NQ):
    hsn_ref[q, :, :] = dis * z[:, q * QW:(q + 1) * QW]


_tc_b = pl.pallas_call(
    _tc_b_body,
    grid=(GRID,),
    in_specs=[
        pl.BlockSpec((NQ, R, QW), lambda i: (0, i, 0)),
        pl.BlockSpec((NQ, R, QW), lambda i: (0, i, 0)),
        pl.BlockSpec((NQ, R, QW), lambda i: (0, i, 0)),
        pl.BlockSpec((R, QW), lambda i: (i, 0)),
        pl.BlockSpec((1, D), lambda i: (0, 0)),
        pl.BlockSpec((1, D), lambda i: (0, 0)),
        pl.BlockSpec((1, D), lambda i: (0, 0)),
        pl.BlockSpec((D, D), lambda i: (0, 0)),
    ],
    out_specs=[
        pl.BlockSpec((NQ, R, QW), lambda i: (0, i, 0)),
        pl.BlockSpec((NQ, R, QW), lambda i: (0, i, 0)),
    ],
    out_shape=[
        jax.ShapeDtypeStruct((NQ, N, QW), jnp.float32),
        jax.ShapeDtypeStruct((NQ, N, QW), jnp.float32),
    ],
)


def _tc_c_body(acc_ref, hs_ref, res_ref, dis_ref, b_ref, g_ref, be_ref,
               h_ref, gs_ref):
  ys, _ = _layer_core(acc_ref, hs_ref, res_ref, dis_ref, b_ref, g_ref, be_ref)
  for q in range(NQ):
    h_ref[:, pl.ds(q * QW, QW)] = ys[q]
  part = jnp.concatenate(
      [jnp.sum(ys[q], axis=0, keepdims=True) for q in range(NQ)], axis=1)

  @pl.when(pl.program_id(0) == 0)
  def _():
    gs_ref[...] = jnp.zeros((1, D), jnp.float32)

  gs_ref[...] += part * (1.0 / N)


_tc_c = pl.pallas_call(
    _tc_c_body,
    grid=(GRID,),
    in_specs=[
        pl.BlockSpec((NQ, R, QW), lambda i: (0, i, 0)),
        pl.BlockSpec((NQ, R, QW), lambda i: (0, i, 0)),
        pl.BlockSpec((NQ, R, QW), lambda i: (0, i, 0)),
        pl.BlockSpec((R, QW), lambda i: (i, 0)),
        pl.BlockSpec((1, D), lambda i: (0, 0)),
        pl.BlockSpec((1, D), lambda i: (0, 0)),
        pl.BlockSpec((1, D), lambda i: (0, 0)),
    ],
    out_specs=[
        pl.BlockSpec((R, D), lambda i: (i, 0)),
        pl.BlockSpec((1, D), lambda i: (0, 0)),
    ],
    out_shape=[
        jax.ShapeDtypeStruct((N, D), jnp.float32),
        jax.ShapeDtypeStruct((1, D), jnp.float32),
    ],
)


def kernel(x, edge_index, proj_W, proj_b, ln_g, ln_b,
           W0, b0, g0, be0, W1, b1, g1, be1, W2, b2, g2, be2):
  npad = NROWP * EB - E
  srcb = jnp.concatenate(
      [edge_index[0], jnp.zeros((npad,), jnp.int32)]).reshape(NROWP, EB)
  dstb = jnp.concatenate(
      [edge_index[1], jnp.full((npad,), N, jnp.int32)]).reshape(NROWP, EB)
  deg16 = _sc_deg(dstb)
  h0q, hs0q, dis2 = _tc_a(x, proj_W, proj_b.reshape(1, D), ln_g.reshape(1, D),
                          ln_b.reshape(1, D), W0, deg16)
  hq, hsq = h0q, hs0q
  layer_params = ((b0, g0, be0), (b1, g1, be1), (b2, g2, be2))
  next_w = (W1, W2)
  for i in range(2):
    accq = _sc_scatter(hsq, srcb, dstb)
    b, g, be = layer_params[i]
    hq, hsq = _tc_b(accq, hsq, hq, dis2, b.reshape(1, D), g.reshape(1, D),
                    be.reshape(1, D), next_w[i])
  accq = _sc_scatter(hsq, srcb, dstb)
  b, g, be = layer_params[2]
  h, gs = _tc_c(accq, hsq, hq, dis2, b.reshape(1, D), g.reshape(1, D),
                be.reshape(1, D))
  return h, gs.reshape(D)
